# Initial kernel scaffold; baseline (speedup 1.0000x reference)
import jax
import jax.numpy as jnp
from jax.experimental import pallas as pl


def kernel(x, edge_index, gamma0, beta0, W1, b1, gamma1, beta1, Wrel2, Wroot2, b2, gamma2, beta2, Wrel3, Wroot3, b3, gamma3, beta3, Wrel4, Wroot4, b4, gamma4, beta4):
    raise NotImplementedError("write your pallas kernel here")


# speedup vs baseline: 6.6825x; 6.6825x over previous
import functools

import jax
import jax.numpy as jnp
from jax import lax
from jax.experimental import pallas as pl
from jax.experimental.pallas import tpu as pltpu
from jax.experimental.pallas import tpu_sc as plsc

N = 10000
E = 320000
EPS = 1e-5

NC = 2
NS = 16
NW = NC * NS
CHUNK = 128
CPT = 79
EPT = CPT * CHUNK
EP = NW * EPT
NP = 10112
RPT = NP // NS

_ZCHUNKS = ((0, 128), (128, 128), (256, 128), (384, 128), (512, RPT - 512))


def _zero_vmem_2d(buf, ncols16):
    z = jnp.zeros((16,), jnp.float32)

    def body(i, c):
        for j in range(ncols16):
            buf[i, pl.ds(16 * j, 16)] = z
        return c

    lax.fori_loop(0, 128, body, 0)


def _segsum_body(h_hbm, src_hbm, dst_hbm, out_hbm, acc, sidx, didx, rows,
                 zbuf, sem):
    cid = lax.axis_index("c")
    sid = lax.axis_index("s")

    _zero_vmem_2d(zbuf, 8)
    rbase = sid * RPT
    for off, n in _ZCHUNKS:
        pltpu.sync_copy(zbuf.at[pl.ds(0, n)], acc.at[pl.ds(rbase + off, n)])
    plsc.subcore_barrier()

    ebase = (cid * NS + sid) * EPT

    def body(t, c):
        off = ebase + t * CHUNK
        pltpu.sync_copy(src_hbm.at[pl.ds(off, CHUNK)], sidx)
        pltpu.sync_copy(dst_hbm.at[pl.ds(off, CHUNK)], didx)
        pltpu.async_copy(h_hbm.at[sidx], rows, sem).wait()
        pltpu.sync_copy(rows, acc.at[didx], add=True)
        return c

    lax.fori_loop(0, CPT, body, 0)
    plsc.subcore_barrier()
    pltpu.sync_copy(acc.at[pl.ds(rbase, RPT)],
                    out_hbm.at[cid, pl.ds(rbase, RPT)])


def _sc_segsum(h_pad, src_p, dst_p):
    mesh = plsc.VectorSubcoreMesh(core_axis_name="c", subcore_axis_name="s")
    return pl.kernel(
        _segsum_body,
        out_type=jax.ShapeDtypeStruct((NC, NP, 128), jnp.float32),
        mesh=mesh,
        scratch_types=[
            pltpu.VMEM_SHARED((NP, 128), jnp.float32),
            pltpu.VMEM((CHUNK,), jnp.int32),
            pltpu.VMEM((CHUNK,), jnp.int32),
            pltpu.VMEM((CHUNK, 128), jnp.float32),
            pltpu.VMEM((128, 128), jnp.float32),
            pltpu.SemaphoreType.DMA,
        ],
    )(h_pad, src_p, dst_p)


def _deg_body(dst_hbm, out_hbm, acc, didx, ones, zbuf):
    cid = lax.axis_index("c")
    sid = lax.axis_index("s")

    _zero_vmem_2d(zbuf, 1)
    ov = jnp.ones((16,), jnp.float32)

    def fill(i, c):
        ones[i, :] = ov
        return c

    lax.fori_loop(0, 128, fill, 0)

    rbase = sid * RPT
    for off, n in _ZCHUNKS:
        pltpu.sync_copy(zbuf.at[pl.ds(0, n)], acc.at[pl.ds(rbase + off, n)])
    plsc.subcore_barrier()

    ebase = (cid * NS + sid) * EPT

    def body(t, c):
        off = ebase + t * CHUNK
        pltpu.sync_copy(dst_hbm.at[pl.ds(off, CHUNK)], didx)
        pltpu.sync_copy(ones, acc.at[didx], add=True)
        return c

    lax.fori_loop(0, CPT, body, 0)
    plsc.subcore_barrier()
    pltpu.sync_copy(acc.at[pl.ds(rbase, RPT)],
                    out_hbm.at[cid, pl.ds(rbase, RPT)])


def _sc_deg(dst_p):
    mesh = plsc.VectorSubcoreMesh(core_axis_name="c", subcore_axis_name="s")
    return pl.kernel(
        _deg_body,
        out_type=jax.ShapeDtypeStruct((NC, NP, 16), jnp.float32),
        mesh=mesh,
        scratch_types=[
            pltpu.VMEM_SHARED((NP, 16), jnp.float32),
            pltpu.VMEM((CHUNK,), jnp.int32),
            pltpu.VMEM((CHUNK, 16), jnp.float32),
            pltpu.VMEM((128, 16), jnp.float32),
        ],
    )(dst_p)


def _batch_norm(h, gamma, beta):
    m = jnp.mean(h, axis=0, keepdims=True)
    hc = h - m
    v = jnp.mean(hc * hc, axis=0, keepdims=True)
    return hc * lax.rsqrt(v + EPS) * gamma[None, :] + beta[None, :]


def _dinv_from_parts(degp_ref):
    degp = degp_ref[...]
    deg = degp[0, :, 0:1] + degp[1, :, 0:1] + 1.0
    return lax.rsqrt(deg)


def _tc1_body(x_ref, w_ref, g_ref, b_ref, degp_ref, hs_ref):
    x = x_ref[...]
    xn = _batch_norm(x, g_ref[...], b_ref[...])
    h1 = jnp.dot(xn, w_ref[...], preferred_element_type=jnp.float32)
    dinv = _dinv_from_parts(degp_ref)
    hs_ref[pl.ds(0, N), :] = h1 * dinv[0:N]
    hs_ref[pl.ds(N, NP - N), :] = jnp.zeros((NP - N, 128), jnp.float32)


def _tc1(x, w1, g0, b0, degp):
    return pl.pallas_call(
        _tc1_body,
        out_shape=jax.ShapeDtypeStruct((NP, 128), jnp.float32),
    )(x, w1, g0, b0, degp)


def _tc2_body(segp_ref, hs_ref, degp_ref, b_ref, g1_ref, be1_ref, out_ref):
    segp = segp_ref[...]
    seg = segp[0, 0:N, :] + segp[1, 0:N, :]
    dinv = _dinv_from_parts(degp_ref)[0:N]
    hs = hs_ref[pl.ds(0, N), :]
    h = jax.nn.relu(dinv * (seg + hs) + b_ref[...][None, :])
    out_ref[pl.ds(0, N), :] = _batch_norm(h, g1_ref[...], be1_ref[...])
    out_ref[pl.ds(N, NP - N), :] = jnp.zeros((NP - N, 128), jnp.float32)


def _tc2(segp, hs, degp, b1, g1, be1):
    return pl.pallas_call(
        _tc2_body,
        out_shape=jax.ShapeDtypeStruct((NP, 128), jnp.float32),
    )(segp, hs, degp, b1, g1, be1)


def _tc_graph_body(aggp_ref, x_ref, wrel_ref, wroot_ref, b_ref, g_ref,
                   be_ref, out_ref, *, hout, pad_out):
    aggp = aggp_ref[...]
    agg = aggp[0, 0:N, :] + aggp[1, 0:N, :]
    x = x_ref[pl.ds(0, N), :]
    y = (jnp.dot(agg, wrel_ref[...], preferred_element_type=jnp.float32)
         + jnp.dot(x, wroot_ref[...], preferred_element_type=jnp.float32)
         + b_ref[...][None, :])
    h = _batch_norm(jax.nn.relu(y), g_ref[...], be_ref[...])
    if pad_out:
        out_ref[pl.ds(0, N), :] = h
        out_ref[pl.ds(N, NP - N), :] = jnp.zeros((NP - N, hout), jnp.float32)
    else:
        out_ref[...] = h


def _tc_graph(aggp, x, wrel, wroot, b, g, be, hout, pad_out):
    nrows = NP if pad_out else N
    return pl.pallas_call(
        functools.partial(_tc_graph_body, hout=hout, pad_out=pad_out),
        out_shape=jax.ShapeDtypeStruct((nrows, hout), jnp.float32),
    )(aggp, x, wrel, wroot, b, g, be)


def kernel(x, edge_index, gamma0, beta0, W1, b1, gamma1, beta1, Wrel2,
           Wroot2, b2, gamma2, beta2, Wrel3, Wroot3, b3, gamma3, beta3,
           Wrel4, Wroot4, b4, gamma4, beta4):
    pad = jnp.full((EP - E,), N, dtype=jnp.int32)
    src_p = jnp.concatenate([edge_index[0], pad])
    dst_p = jnp.concatenate([edge_index[1], pad])

    degp = _sc_deg(dst_p)
    hs = _tc1(x, W1, gamma0, beta0, degp)
    segp = _sc_segsum(hs, src_p, dst_p)
    x2 = _tc2(segp, hs, degp, b1, gamma1, beta1)
    aggp = _sc_segsum(x2, src_p, dst_p)
    x3 = _tc_graph(aggp, x2, Wrel2, Wroot2, b2, gamma2, beta2, 128, True)
    aggp = _sc_segsum(x3, src_p, dst_p)
    x4 = _tc_graph(aggp, x3, Wrel3, Wroot3, b3, gamma3, beta3, 128, True)
    aggp = _sc_segsum(x4, src_p, dst_p)
    return _tc_graph(aggp, x4, Wrel4, Wroot4, b4, gamma4, beta4, 64, False)
